# baseline (speedup 1.0000x reference)
import jax
import jax.numpy as jnp
from jax import lax
from jax.experimental import pallas as pl
from jax.experimental.pallas import tpu as pltpu
from jax.experimental.pallas import tpu_sc as plsc

N_USERS = 1000000
N_ITEMS = 1000000
EMBED_DIM = 32
BATCH = 16384

NC = 2
NS = 16
L = 16
NW = NC * NS
B_PER_W = BATCH // NW
IDX_CHUNK = 128
N_CHUNKS = B_PER_W // IDX_CHUNK
G_PER_CHUNK = IDX_CHUNK // L

RP_ROWS = 512
RP_WORDS = RP_ROWS * EMBED_DIM
MAIN_T = 61
FULL_CHUNKS = N_USERS // RP_ROWS
TAIL_BASE = FULL_CHUNKS * RP_ROWS
TAIL_ROWS = N_USERS - TAIL_BASE


def _repack_kernel(user_table, item_table, uflat, iflat,
                   bin0, bout, sem_in, sem_out):
    wid = lax.axis_index("s") * NC + lax.axis_index("c")

    def bridge(nrows):
        def body(r4, _):
            for u in range(4):
                r = r4 * 4 + u
                v0 = bin0[r, pl.ds(0, L)]
                v1 = bin0[r, pl.ds(L, L)]
                bout[pl.ds(r * EMBED_DIM, L)] = v0
                bout[pl.ds(r * EMBED_DIM + L, L)] = v1
            return _

        lax.fori_loop(0, nrows // 4, body, None)

    def run_table(table, out):
        def wait_out():
            pltpu.make_async_copy(
                bout, out.at[pl.ds(0, RP_WORDS)], sem_out).wait()

        for t in range(MAIN_T):
            pltpu.sync_copy(
                table.at[pl.ds((wid + 32 * t) * RP_ROWS, RP_ROWS)], bin0)
            if t >= 1:
                wait_out()
            bridge(RP_ROWS)
            pltpu.async_copy(
                bout, out.at[pl.ds((wid + 32 * t) * RP_WORDS, RP_WORDS)],
                sem_out)
        wait_out()

        @pl.when(wid == 0)
        def _extra():
            pltpu.sync_copy(table.at[pl.ds(1952 * RP_ROWS, RP_ROWS)], bin0)
            bridge(RP_ROWS)
            pltpu.sync_copy(bout, out.at[pl.ds(1952 * RP_WORDS, RP_WORDS)])

        @pl.when(wid == 1)
        def _tail():
            pltpu.sync_copy(table.at[pl.ds(TAIL_BASE, TAIL_ROWS)],
                            bin0.at[pl.ds(0, TAIL_ROWS)])
            bridge(TAIL_ROWS)
            pltpu.sync_copy(
                bout.at[pl.ds(0, TAIL_ROWS * EMBED_DIM)],
                out.at[pl.ds(TAIL_BASE * EMBED_DIM, TAIL_ROWS * EMBED_DIM)])

    run_table(user_table, uflat)
    run_table(item_table, iflat)


def _score_kernel(user_idx_hbm, item_idx_hbm, uflat, iflat,
                  user_bias, item_bias, gb_hbm, out_hbm,
                  idx_u, idx_i, el_u, el_i, cols_u, cols_i,
                  bias_u, bias_i, gb_v, out_v, sem, sem_b):
    wid = lax.axis_index("s") * NC + lax.axis_index("c")
    base_blk = wid * N_CHUNKS

    pltpu.sync_copy(user_idx_hbm.at[pl.ds(base_blk, N_CHUNKS)], idx_u)
    pltpu.sync_copy(item_idx_hbm.at[pl.ds(base_blk, N_CHUNKS)], idx_i)
    pltpu.sync_copy(gb_hbm, gb_v)

    bias_copies = []
    for j in range(N_CHUNKS):
        s = pl.ds(j * IDX_CHUNK, IDX_CHUNK)
        bias_copies.append(pltpu.async_copy(
            user_bias.at[idx_u.at[j]], bias_u.at[s], sem_b))
        bias_copies.append(pltpu.async_copy(
            item_bias.at[idx_i.at[j]], bias_i.at[s], sem_b))

    for c in bias_copies:
        c.wait()
    gb = gb_v[...]

    for j in range(N_CHUNKS):
        def build(g, _, j=j):
            s = pl.ds(g * L, L)
            bu = idx_u[j, s] << 5
            bi = idx_i[j, s] << 5
            for d in range(EMBED_DIM):
                el_u[d, s] = bu + d
                el_i[d, s] = bi + d
            return _

        lax.fori_loop(0, G_PER_CHUNK, build, None)

        copies = []
        for d in range(EMBED_DIM):
            copies.append(pltpu.async_copy(
                uflat.at[el_u.at[d]], cols_u.at[d], sem))
            copies.append(pltpu.async_copy(
                iflat.at[el_i.at[d]], cols_i.at[d], sem))
        for c in copies:
            c.wait()

        def body(g, _, j=j):
            s = pl.ds(g * L, L)
            acc = None
            for d in range(EMBED_DIM):
                prod = cols_u[d, s] * cols_i[d, s]
                acc = prod if acc is None else acc + prod
            so = pl.ds(j * IDX_CHUNK + g * L, L)
            p = acc + bias_u[so] + bias_i[so] + gb
            out_v[so] = 1.0 / (1.0 + jnp.exp(-p))
            return _

        lax.fori_loop(0, G_PER_CHUNK, body, None)

    pltpu.sync_copy(out_v, out_hbm.at[pl.ds(wid * B_PER_W, B_PER_W)])


def kernel(user_idx, item_idx, user_table, item_table, user_bias_table,
           item_bias_table, global_bias):
    repack = pl.kernel(
        _repack_kernel,
        mesh=plsc.VectorSubcoreMesh(core_axis_name="c", subcore_axis_name="s"),
        compiler_params=pltpu.CompilerParams(needs_layout_passes=False),
        out_type=(jax.ShapeDtypeStruct((N_USERS * EMBED_DIM,), jnp.float32),
                  jax.ShapeDtypeStruct((N_ITEMS * EMBED_DIM,), jnp.float32)),
        scratch_types=[
            pltpu.VMEM((RP_ROWS, EMBED_DIM), jnp.float32),
            pltpu.VMEM((RP_WORDS,), jnp.float32),
            pltpu.SemaphoreType.DMA,
            pltpu.SemaphoreType.DMA,
        ],
    )
    score = pl.kernel(
        _score_kernel,
        mesh=plsc.VectorSubcoreMesh(core_axis_name="c", subcore_axis_name="s"),
        compiler_params=pltpu.CompilerParams(needs_layout_passes=False),
        out_type=jax.ShapeDtypeStruct((BATCH,), jnp.float32),
        scratch_types=[
            pltpu.VMEM((N_CHUNKS, IDX_CHUNK), jnp.int32),
            pltpu.VMEM((N_CHUNKS, IDX_CHUNK), jnp.int32),
            pltpu.VMEM((EMBED_DIM, IDX_CHUNK), jnp.int32),
            pltpu.VMEM((EMBED_DIM, IDX_CHUNK), jnp.int32),
            pltpu.VMEM((EMBED_DIM, IDX_CHUNK), jnp.float32),
            pltpu.VMEM((EMBED_DIM, IDX_CHUNK), jnp.float32),
            pltpu.VMEM((B_PER_W,), jnp.float32),
            pltpu.VMEM((B_PER_W,), jnp.float32),
            pltpu.VMEM((L,), jnp.float32),
            pltpu.VMEM((B_PER_W,), jnp.float32),
            pltpu.SemaphoreType.DMA,
            pltpu.SemaphoreType.DMA,
        ],
    )
    uidx = user_idx.astype(jnp.int32).reshape(BATCH // IDX_CHUNK, IDX_CHUNK)
    iidx = item_idx.astype(jnp.int32).reshape(BATCH // IDX_CHUNK, IDX_CHUNK)
    gb16 = jnp.broadcast_to(global_bias.astype(jnp.float32), (L,))
    uflat, iflat = repack(user_table, item_table)
    return score(uidx, iidx, uflat, iflat,
                 user_bias_table.reshape(N_USERS),
                 item_bias_table.reshape(N_ITEMS),
                 gb16)
